# Initial kernel scaffold; baseline (speedup 1.0000x reference)
import jax
import jax.numpy as jnp
from jax.experimental import pallas as pl


def kernel(x, edge_index, W1, b1, W3, b3, W2, b2):
    raise NotImplementedError("write your pallas kernel here")


# speedup vs baseline: 9.8527x; 9.8527x over previous
import functools

import jax
import jax.numpy as jnp
from jax import lax
from jax.experimental import pallas as pl
from jax.experimental.pallas import tpu as pltpu
from jax.experimental.pallas import tpu_sc as plsc

N_NODES = 10000
N_EDGES = 320000
NP = 10240
NC = 2
NS = 16
NW = NC * NS
CHUNK = 128
RPW = 80
EP = NW * RPW * CHUNK
BR = 256

_mesh = plsc.VectorSubcoreMesh(core_axis_name="c", subcore_axis_name="s")
_sc_params = pltpu.CompilerParams(needs_layout_passes=False,
                                  use_tc_tiling_on_sc=False)


@functools.partial(
    pl.kernel,
    out_type=jax.ShapeDtypeStruct((NW * NP,), jnp.float32),
    mesh=_mesh,
    compiler_params=_sc_params,
    scratch_types=[
        pltpu.VMEM((RPW, CHUNK), jnp.int32),
        pltpu.VMEM((NP,), jnp.float32),
    ],
)
def _deg_kernel(dst2d, out, dstv, degl):
    c = lax.axis_index("c")
    s = lax.axis_index("s")
    wid = s * NC + c
    zeros16 = jnp.zeros((16,), jnp.float32)
    ones16 = jnp.ones((16,), jnp.float32)

    def zero_body(i, carry):
        degl[pl.ds(i * 16, 16)] = zeros16
        return carry
    lax.fori_loop(0, NP // 16, zero_body, 0)

    pltpu.sync_copy(dst2d.at[pl.ds(wid * RPW, RPW)], dstv)

    def hist_body(j, carry):
        for l in range(CHUNK // 16):
            idx = dstv[j, pl.ds(l * 16, 16)]
            plsc.addupdate_scatter(degl, [idx], ones16)
        return carry
    lax.fori_loop(0, RPW, hist_body, 0)

    pltpu.sync_copy(degl, out.at[pl.ds(wid * NP, NP)])


def _make_agg_kernel(d):
    @functools.partial(
        pl.kernel,
        out_type=jax.ShapeDtypeStruct((NC, NP, d), jnp.float32),
        mesh=_mesh,
        compiler_params=_sc_params,
        scratch_types=[
            pltpu.VMEM((RPW, CHUNK), jnp.int32),
            pltpu.VMEM((RPW, CHUNK), jnp.int32),
            pltpu.VMEM((CHUNK, d), jnp.float32),
            pltpu.VMEM_SHARED((NP, d), jnp.float32),
            pltpu.SemaphoreType.DMA,
        ],
    )
    def agg(g_hbm, src2d, dst2d, out, srcv, dstv, buf, shared, sem):
        c = lax.axis_index("c")
        s = lax.axis_index("s")
        wid = s * NC + c
        zeros16 = jnp.zeros((16,), jnp.float32)

        def zero_body(i, carry):
            for l in range(d // 16):
                buf[i, pl.ds(l * 16, 16)] = zeros16
            return carry
        lax.fori_loop(0, CHUNK, zero_body, 0)

        for k in range(NP // NS // CHUNK):
            pltpu.sync_copy(buf, shared.at[pl.ds(s * (NP // NS) + k * CHUNK, CHUNK)])
        plsc.subcore_barrier()

        pltpu.sync_copy(src2d.at[pl.ds(wid * RPW, RPW)], srcv)
        pltpu.sync_copy(dst2d.at[pl.ds(wid * RPW, RPW)], dstv)

        def chunk_body(j, carry):
            pltpu.async_copy(g_hbm.at[srcv.at[j]], buf, sem).wait()
            pltpu.sync_copy(buf, shared.at[dstv.at[j]], add=True)
            return carry
        lax.fori_loop(0, RPW, chunk_body, 0)

        plsc.subcore_barrier()
        for k in range(NP // NS // CHUNK):
            r0 = s * (NP // NS) + k * CHUNK
            pltpu.sync_copy(shared.at[pl.ds(r0, CHUNK)], out.at[c, pl.ds(r0, CHUNK)])

    return agg


_agg64 = _make_agg_kernel(64)
_agg128 = _make_agg_kernel(128)


def _dinv_block(p_block):
    return lax.rsqrt(jnp.sum(p_block, axis=0) + 1.0)[:, None]


def _tc_pre(x, w, degp, dout):
    def body(x_ref, w_ref, p_ref, h_ref, g_ref):
        dinv = _dinv_block(p_ref[...])
        h = jnp.dot(x_ref[...], w_ref[...], preferred_element_type=jnp.float32)
        h_ref[...] = h
        g_ref[...] = h * dinv
    din = x.shape[1]
    return pl.pallas_call(
        body,
        grid=(NP // BR,),
        in_specs=[
            pl.BlockSpec((BR, din), lambda i: (i, 0)),
            pl.BlockSpec((din, dout), lambda i: (0, 0)),
            pl.BlockSpec((NW, BR), lambda i: (0, i)),
        ],
        out_specs=[pl.BlockSpec((BR, dout), lambda i: (i, 0))] * 2,
        out_shape=[jax.ShapeDtypeStruct((NP, dout), jnp.float32)] * 2,
    )(x, w, degp)


def _tc_mid(h, a0, a1, degp, b, w, dout):
    dp = h.shape[1]
    def body(h_ref, a0_ref, a1_ref, p_ref, b_ref, w_ref, hn_ref, gn_ref):
        dinv = _dinv_block(p_ref[...])
        xn = jnp.maximum(
            dinv * (a0_ref[...] + a1_ref[...] + dinv * h_ref[...]) + b_ref[...],
            0.0)
        hn = jnp.dot(xn, w_ref[...], preferred_element_type=jnp.float32)
        hn_ref[...] = hn
        gn_ref[...] = hn * dinv
    return pl.pallas_call(
        body,
        grid=(NP // BR,),
        in_specs=[
            pl.BlockSpec((BR, dp), lambda i: (i, 0)),
            pl.BlockSpec((BR, dp), lambda i: (i, 0)),
            pl.BlockSpec((BR, dp), lambda i: (i, 0)),
            pl.BlockSpec((NW, BR), lambda i: (0, i)),
            pl.BlockSpec((1, dp), lambda i: (0, 0)),
            pl.BlockSpec((dp, dout), lambda i: (0, 0)),
        ],
        out_specs=[pl.BlockSpec((BR, dout), lambda i: (i, 0))] * 2,
        out_shape=[jax.ShapeDtypeStruct((NP, dout), jnp.float32)] * 2,
    )(h, a0, a1, degp, b, w)


def _tc_post(h, a0, a1, degp, b):
    dp = h.shape[1]
    def body(h_ref, a0_ref, a1_ref, p_ref, b_ref, o_ref):
        dinv = _dinv_block(p_ref[...])
        o_ref[...] = jnp.maximum(
            dinv * (a0_ref[...] + a1_ref[...] + dinv * h_ref[...]) + b_ref[...],
            0.0)
    return pl.pallas_call(
        body,
        grid=(NP // BR,),
        in_specs=[
            pl.BlockSpec((BR, dp), lambda i: (i, 0)),
            pl.BlockSpec((BR, dp), lambda i: (i, 0)),
            pl.BlockSpec((BR, dp), lambda i: (i, 0)),
            pl.BlockSpec((NW, BR), lambda i: (0, i)),
            pl.BlockSpec((1, dp), lambda i: (0, 0)),
        ],
        out_specs=pl.BlockSpec((BR, dp), lambda i: (i, 0)),
        out_shape=jax.ShapeDtypeStruct((NP, dp), jnp.float32),
    )(h, a0, a1, degp, b)


def kernel(x, edge_index, W1, b1, W3, b3, W2, b2):
    src = edge_index[0].astype(jnp.int32)
    dst = edge_index[1].astype(jnp.int32)
    pad_e = EP - N_EDGES
    src2d = jnp.concatenate(
        [src, jnp.zeros((pad_e,), jnp.int32)]).reshape(EP // CHUNK, CHUNK)
    dst2d = jnp.concatenate(
        [dst, jnp.full((pad_e,), N_NODES, jnp.int32)]).reshape(EP // CHUNK, CHUNK)
    xp = jnp.pad(x, ((0, NP - N_NODES), (0, 0)))

    degp = _deg_kernel(dst2d).reshape(NW, NP)

    h1, g1 = _tc_pre(xp, W1, degp, 64)
    a1 = _agg64(g1, src2d, dst2d)
    h2, g2 = _tc_mid(h1, a1[0], a1[1], degp, b1.reshape(1, -1), W3, 64)
    a2 = _agg64(g2, src2d, dst2d)
    h3, g3 = _tc_mid(h2, a2[0], a2[1], degp, b3.reshape(1, -1), W2, 128)
    a3 = _agg128(g3, src2d, dst2d)
    out = _tc_post(h3, a3[0], a3[1], degp, b2.reshape(1, -1))
    return out[:N_NODES]


# speedup vs baseline: 10.8599x; 1.1022x over previous
import functools

import jax
import jax.numpy as jnp
from jax import lax
from jax.experimental import pallas as pl
from jax.experimental.pallas import tpu as pltpu
from jax.experimental.pallas import tpu_sc as plsc

N_NODES = 10000
N_EDGES = 320000
NP = 10240
NC = 2
NS = 16
NW = NC * NS
CHUNK = 128
RPW = 80
EP = NW * RPW * CHUNK
BR = 256

_mesh = plsc.VectorSubcoreMesh(core_axis_name="c", subcore_axis_name="s")
_sc_params = pltpu.CompilerParams(needs_layout_passes=False,
                                  use_tc_tiling_on_sc=False)


@functools.partial(
    pl.kernel,
    out_type=jax.ShapeDtypeStruct((NW * NP,), jnp.float32),
    mesh=_mesh,
    compiler_params=_sc_params,
    scratch_types=[
        pltpu.VMEM((RPW, CHUNK), jnp.int32),
        pltpu.VMEM((NP,), jnp.float32),
    ],
)
def _deg_kernel(dst2d, out, dstv, degl):
    c = lax.axis_index("c")
    s = lax.axis_index("s")
    wid = s * NC + c
    zeros16 = jnp.zeros((16,), jnp.float32)
    ones16 = jnp.ones((16,), jnp.float32)

    def zero_body(i, carry):
        degl[pl.ds(i * 16, 16)] = zeros16
        return carry
    lax.fori_loop(0, NP // 16, zero_body, 0)

    pltpu.sync_copy(dst2d.at[pl.ds(wid * RPW, RPW)], dstv)

    def hist_body(j, carry):
        for l in range(CHUNK // 16):
            idx = dstv[j, pl.ds(l * 16, 16)]
            plsc.addupdate_scatter(degl, [idx], ones16)
        return carry
    lax.fori_loop(0, RPW, hist_body, 0)

    pltpu.sync_copy(degl, out.at[pl.ds(wid * NP, NP)])


def _make_agg_kernel(d, pipelined=True):
    @functools.partial(
        pl.kernel,
        out_type=jax.ShapeDtypeStruct((NC, NP, d), jnp.float32),
        mesh=_mesh,
        compiler_params=_sc_params,
        scratch_types=[
            pltpu.VMEM((RPW, CHUNK), jnp.int32),
            pltpu.VMEM((RPW, CHUNK), jnp.int32),
            pltpu.VMEM((CHUNK, d), jnp.float32),
            pltpu.VMEM((CHUNK, d), jnp.float32),
            pltpu.VMEM_SHARED((NP, d), jnp.float32),
            pltpu.SemaphoreType.DMA,
            pltpu.SemaphoreType.DMA,
        ],
    )
    def agg(g_hbm, src2d, dst2d, out, srcv, dstv, buf0, buf1, shared, sem0, sem1):
        c = lax.axis_index("c")
        s = lax.axis_index("s")
        wid = s * NC + c
        zeros16 = jnp.zeros((16,), jnp.float32)

        def zero_body(i, carry):
            for l in range(d // 16):
                buf0[i, pl.ds(l * 16, 16)] = zeros16
            return carry
        lax.fori_loop(0, CHUNK, zero_body, 0)

        for k in range(NP // NS // CHUNK):
            pltpu.sync_copy(buf0, shared.at[pl.ds(s * (NP // NS) + k * CHUNK, CHUNK)])
        plsc.subcore_barrier()

        pltpu.sync_copy(src2d.at[pl.ds(wid * RPW, RPW)], srcv)
        pltpu.sync_copy(dst2d.at[pl.ds(wid * RPW, RPW)], dstv)

        if pipelined:
            pltpu.async_copy(g_hbm.at[srcv.at[0]], buf0, sem0)

            def chunk_body(i, carry):
                base = i * 2
                cp1 = pltpu.async_copy(g_hbm.at[srcv.at[base + 1]], buf1, sem1)
                pltpu.make_async_copy(g_hbm.at[srcv.at[base]], buf0, sem0).wait()
                pltpu.sync_copy(buf0, shared.at[dstv.at[base]], add=True)

                @pl.when(base + 2 < RPW)
                def _():
                    pltpu.async_copy(g_hbm.at[srcv.at[base + 2]], buf0, sem0)
                cp1.wait()
                pltpu.sync_copy(buf1, shared.at[dstv.at[base + 1]], add=True)
                return carry
            lax.fori_loop(0, RPW // 2, chunk_body, 0)
        else:
            def chunk_body(j, carry):
                pltpu.async_copy(g_hbm.at[srcv.at[j]], buf0, sem0).wait()
                pltpu.sync_copy(buf0, shared.at[dstv.at[j]], add=True)
                return carry
            lax.fori_loop(0, RPW, chunk_body, 0)

        plsc.subcore_barrier()
        for k in range(NP // NS // CHUNK):
            r0 = s * (NP // NS) + k * CHUNK
            pltpu.sync_copy(shared.at[pl.ds(r0, CHUNK)], out.at[c, pl.ds(r0, CHUNK)])

    return agg


_agg64 = _make_agg_kernel(64)


def _dinv_block(p_block):
    return lax.rsqrt(jnp.sum(p_block, axis=0) + 1.0)[:, None]


def _tc_pre(x, w, degp, dout):
    def body(x_ref, w_ref, p_ref, h_ref, g_ref):
        dinv = _dinv_block(p_ref[...])
        h = jnp.dot(x_ref[...], w_ref[...], preferred_element_type=jnp.float32)
        h_ref[...] = h
        g_ref[...] = h * dinv
    din = x.shape[1]
    return pl.pallas_call(
        body,
        grid=(NP // BR,),
        in_specs=[
            pl.BlockSpec((BR, din), lambda i: (i, 0)),
            pl.BlockSpec((din, dout), lambda i: (0, 0)),
            pl.BlockSpec((NW, BR), lambda i: (0, i)),
        ],
        out_specs=[pl.BlockSpec((BR, dout), lambda i: (i, 0))] * 2,
        out_shape=[jax.ShapeDtypeStruct((NP, dout), jnp.float32)] * 2,
    )(x, w, degp)


def _tc_mid(h, a0, a1, degp, b, w, dout, split_g=False):
    dp = h.shape[1]
    def body(h_ref, a0_ref, a1_ref, p_ref, b_ref, w_ref, hn_ref, *g_refs):
        dinv = _dinv_block(p_ref[...])
        xn = jnp.maximum(
            dinv * (a0_ref[...] + a1_ref[...] + dinv * h_ref[...]) + b_ref[...],
            0.0)
        hn = jnp.dot(xn, w_ref[...], preferred_element_type=jnp.float32)
        hn_ref[...] = hn
        gn = hn * dinv
        if split_g:
            g_refs[0][...] = gn[:, :dout // 2]
            g_refs[1][...] = gn[:, dout // 2:]
        else:
            g_refs[0][...] = gn
    dg = dout // 2 if split_g else dout
    n_g = 2 if split_g else 1
    return pl.pallas_call(
        body,
        grid=(NP // BR,),
        in_specs=[
            pl.BlockSpec((BR, dp), lambda i: (i, 0)),
            pl.BlockSpec((BR, dp), lambda i: (i, 0)),
            pl.BlockSpec((BR, dp), lambda i: (i, 0)),
            pl.BlockSpec((NW, BR), lambda i: (0, i)),
            pl.BlockSpec((1, dp), lambda i: (0, 0)),
            pl.BlockSpec((dp, dout), lambda i: (0, 0)),
        ],
        out_specs=[pl.BlockSpec((BR, dout), lambda i: (i, 0))]
        + [pl.BlockSpec((BR, dg), lambda i: (i, 0))] * n_g,
        out_shape=[jax.ShapeDtypeStruct((NP, dout), jnp.float32)]
        + [jax.ShapeDtypeStruct((NP, dg), jnp.float32)] * n_g,
    )(h, a0, a1, degp, b, w)


def _tc_post(h, aa0, aa1, ab0, ab1, degp, b):
    dp = h.shape[1]
    dh = dp // 2
    def body(h_ref, aa0_ref, aa1_ref, ab0_ref, ab1_ref, p_ref, b_ref, o_ref):
        dinv = _dinv_block(p_ref[...])
        agg = jnp.concatenate(
            [aa0_ref[...] + aa1_ref[...], ab0_ref[...] + ab1_ref[...]], axis=1)
        o_ref[...] = jnp.maximum(
            dinv * (agg + dinv * h_ref[...]) + b_ref[...], 0.0)
    return pl.pallas_call(
        body,
        grid=(NP // BR,),
        in_specs=[
            pl.BlockSpec((BR, dp), lambda i: (i, 0)),
            pl.BlockSpec((BR, dh), lambda i: (i, 0)),
            pl.BlockSpec((BR, dh), lambda i: (i, 0)),
            pl.BlockSpec((BR, dh), lambda i: (i, 0)),
            pl.BlockSpec((BR, dh), lambda i: (i, 0)),
            pl.BlockSpec((NW, BR), lambda i: (0, i)),
            pl.BlockSpec((1, dp), lambda i: (0, 0)),
        ],
        out_specs=pl.BlockSpec((BR, dp), lambda i: (i, 0)),
        out_shape=jax.ShapeDtypeStruct((NP, dp), jnp.float32),
    )(h, aa0, aa1, ab0, ab1, degp, b)


def kernel(x, edge_index, W1, b1, W3, b3, W2, b2):
    src = edge_index[0].astype(jnp.int32)
    dst = edge_index[1].astype(jnp.int32)
    pad_e = EP - N_EDGES
    src2d = jnp.concatenate(
        [src, jnp.zeros((pad_e,), jnp.int32)]).reshape(EP // CHUNK, CHUNK)
    dst2d = jnp.concatenate(
        [dst, jnp.full((pad_e,), N_NODES, jnp.int32)]).reshape(EP // CHUNK, CHUNK)
    xp = jnp.pad(x, ((0, NP - N_NODES), (0, 0)))

    degp = _deg_kernel(dst2d).reshape(NW, NP)

    h1, g1 = _tc_pre(xp, W1, degp, 64)
    a1 = _agg64(g1, src2d, dst2d)
    h2, g2 = _tc_mid(h1, a1[0], a1[1], degp, b1.reshape(1, -1), W3, 64)
    a2 = _agg64(g2, src2d, dst2d)
    h3, g3a, g3b = _tc_mid(h2, a2[0], a2[1], degp, b3.reshape(1, -1), W2, 128,
                           split_g=True)
    a3a = _agg64(g3a, src2d, dst2d)
    a3b = _agg64(g3b, src2d, dst2d)
    out = _tc_post(h3, a3a[0], a3a[1], a3b[0], a3b[1], degp, b2.reshape(1, -1))
    return out[:N_NODES]


# speedup vs baseline: 21.9019x; 2.0168x over previous
import functools

import jax
import jax.numpy as jnp
from jax import lax
from jax.experimental import pallas as pl
from jax.experimental.pallas import tpu as pltpu
from jax.experimental.pallas import tpu_sc as plsc

N_NODES = 10000
N_EDGES = 320000
NP = 10240
NC = 2
NS = 16
NW = NC * NS
CHUNK = 128
RPW = 80
EP = NW * RPW * CHUNK
BR = 256

_mesh = plsc.VectorSubcoreMesh(core_axis_name="c", subcore_axis_name="s")
_sc_params = pltpu.CompilerParams(needs_layout_passes=False,
                                  use_tc_tiling_on_sc=False)


@functools.partial(
    pl.kernel,
    out_type=jax.ShapeDtypeStruct((NW * NP,), jnp.float32),
    mesh=_mesh,
    compiler_params=_sc_params,
    scratch_types=[
        pltpu.VMEM((RPW, CHUNK), jnp.int32),
        pltpu.VMEM((NP,), jnp.float32),
    ],
)
def _deg_kernel(dst2d, out, dstv, degl):
    c = lax.axis_index("c")
    s = lax.axis_index("s")
    wid = s * NC + c
    zeros16 = jnp.zeros((16,), jnp.float32)
    ones16 = jnp.ones((16,), jnp.float32)

    def zero_body(i, carry):
        degl[pl.ds(i * 16, 16)] = zeros16
        return carry
    lax.fori_loop(0, NP // 16, zero_body, 0)

    pltpu.sync_copy(dst2d.at[pl.ds(wid * RPW, RPW)], dstv)

    def hist_body(j, carry):
        for l in range(CHUNK // 16):
            idx = dstv[j, pl.ds(l * 16, 16)]
            plsc.addupdate_scatter(degl, [idx], ones16)
        return carry
    lax.fori_loop(0, RPW, hist_body, 0)

    pltpu.sync_copy(degl, out.at[pl.ds(wid * NP, NP)])


def _make_agg_kernel(d):
    @functools.partial(
        pl.kernel,
        out_type=jax.ShapeDtypeStruct((NC, NP, d), jnp.float32),
        mesh=_mesh,
        compiler_params=_sc_params,
        scratch_types=[
            pltpu.VMEM((RPW, CHUNK), jnp.int32),
            pltpu.VMEM((RPW, CHUNK), jnp.int32),
            pltpu.VMEM((CHUNK, d), jnp.float32),
            pltpu.VMEM((CHUNK, d), jnp.float32),
            pltpu.VMEM_SHARED((NP, d), jnp.float32),
            pltpu.VMEM_SHARED((NP, d), jnp.float32),
            pltpu.SemaphoreType.DMA,
            pltpu.SemaphoreType.DMA,
        ],
    )
    def agg(g_hbm, src2d, dst2d, out, srcv, dstv, buf0, buf1, gshr, shared,
            sem0, sem1):
        c = lax.axis_index("c")
        s = lax.axis_index("s")
        wid = s * NC + c
        rpt = NP // NS
        zeros16 = jnp.zeros((16,), jnp.float32)

        def zero_body(i, carry):
            for l in range(d // 16):
                buf0[i, pl.ds(l * 16, 16)] = zeros16
            return carry
        lax.fori_loop(0, CHUNK, zero_body, 0)

        pltpu.sync_copy(g_hbm.at[pl.ds(s * rpt, rpt)], gshr.at[pl.ds(s * rpt, rpt)])
        for k in range(rpt // CHUNK):
            pltpu.sync_copy(buf0, shared.at[pl.ds(s * rpt + k * CHUNK, CHUNK)])
        plsc.subcore_barrier()

        pltpu.sync_copy(src2d.at[pl.ds(wid * RPW, RPW)], srcv)
        pltpu.sync_copy(dst2d.at[pl.ds(wid * RPW, RPW)], dstv)

        pltpu.async_copy(gshr.at[srcv.at[0]], buf0, sem0)

        def chunk_body(i, carry):
            base = i * 2
            cp1 = pltpu.async_copy(gshr.at[srcv.at[base + 1]], buf1, sem1)
            pltpu.make_async_copy(gshr.at[srcv.at[base]], buf0, sem0).wait()
            pltpu.sync_copy(buf0, shared.at[dstv.at[base]], add=True)

            @pl.when(base + 2 < RPW)
            def _():
                pltpu.async_copy(gshr.at[srcv.at[base + 2]], buf0, sem0)
            cp1.wait()
            pltpu.sync_copy(buf1, shared.at[dstv.at[base + 1]], add=True)
            return carry
        lax.fori_loop(0, RPW // 2, chunk_body, 0)

        plsc.subcore_barrier()
        for k in range(NP // NS // CHUNK):
            r0 = s * (NP // NS) + k * CHUNK
            pltpu.sync_copy(shared.at[pl.ds(r0, CHUNK)], out.at[c, pl.ds(r0, CHUNK)])

    return agg


_agg64 = _make_agg_kernel(64)


def _dinv_block(p_block):
    return lax.rsqrt(jnp.sum(p_block, axis=0) + 1.0)[:, None]


def _tc_pre(x, w, degp, dout):
    def body(x_ref, w_ref, p_ref, h_ref, g_ref):
        dinv = _dinv_block(p_ref[...])
        h = jnp.dot(x_ref[...], w_ref[...], preferred_element_type=jnp.float32)
        h_ref[...] = h
        g_ref[...] = h * dinv
    din = x.shape[1]
    return pl.pallas_call(
        body,
        grid=(NP // BR,),
        in_specs=[
            pl.BlockSpec((BR, din), lambda i: (i, 0)),
            pl.BlockSpec((din, dout), lambda i: (0, 0)),
            pl.BlockSpec((NW, BR), lambda i: (0, i)),
        ],
        out_specs=[pl.BlockSpec((BR, dout), lambda i: (i, 0))] * 2,
        out_shape=[jax.ShapeDtypeStruct((NP, dout), jnp.float32)] * 2,
    )(x, w, degp)


def _tc_mid(h, a0, a1, degp, b, w, dout, split_g=False):
    dp = h.shape[1]
    def body(h_ref, a0_ref, a1_ref, p_ref, b_ref, w_ref, hn_ref, *g_refs):
        dinv = _dinv_block(p_ref[...])
        xn = jnp.maximum(
            dinv * (a0_ref[...] + a1_ref[...] + dinv * h_ref[...]) + b_ref[...],
            0.0)
        hn = jnp.dot(xn, w_ref[...], preferred_element_type=jnp.float32)
        hn_ref[...] = hn
        gn = hn * dinv
        if split_g:
            g_refs[0][...] = gn[:, :dout // 2]
            g_refs[1][...] = gn[:, dout // 2:]
        else:
            g_refs[0][...] = gn
    dg = dout // 2 if split_g else dout
    n_g = 2 if split_g else 1
    return pl.pallas_call(
        body,
        grid=(NP // BR,),
        in_specs=[
            pl.BlockSpec((BR, dp), lambda i: (i, 0)),
            pl.BlockSpec((BR, dp), lambda i: (i, 0)),
            pl.BlockSpec((BR, dp), lambda i: (i, 0)),
            pl.BlockSpec((NW, BR), lambda i: (0, i)),
            pl.BlockSpec((1, dp), lambda i: (0, 0)),
            pl.BlockSpec((dp, dout), lambda i: (0, 0)),
        ],
        out_specs=[pl.BlockSpec((BR, dout), lambda i: (i, 0))]
        + [pl.BlockSpec((BR, dg), lambda i: (i, 0))] * n_g,
        out_shape=[jax.ShapeDtypeStruct((NP, dout), jnp.float32)]
        + [jax.ShapeDtypeStruct((NP, dg), jnp.float32)] * n_g,
    )(h, a0, a1, degp, b, w)


def _tc_post(h, aa0, aa1, ab0, ab1, degp, b):
    dp = h.shape[1]
    dh = dp // 2
    def body(h_ref, aa0_ref, aa1_ref, ab0_ref, ab1_ref, p_ref, b_ref, o_ref):
        dinv = _dinv_block(p_ref[...])
        agg = jnp.concatenate(
            [aa0_ref[...] + aa1_ref[...], ab0_ref[...] + ab1_ref[...]], axis=1)
        o_ref[...] = jnp.maximum(
            dinv * (agg + dinv * h_ref[...]) + b_ref[...], 0.0)
    return pl.pallas_call(
        body,
        grid=(NP // BR,),
        in_specs=[
            pl.BlockSpec((BR, dp), lambda i: (i, 0)),
            pl.BlockSpec((BR, dh), lambda i: (i, 0)),
            pl.BlockSpec((BR, dh), lambda i: (i, 0)),
            pl.BlockSpec((BR, dh), lambda i: (i, 0)),
            pl.BlockSpec((BR, dh), lambda i: (i, 0)),
            pl.BlockSpec((NW, BR), lambda i: (0, i)),
            pl.BlockSpec((1, dp), lambda i: (0, 0)),
        ],
        out_specs=pl.BlockSpec((BR, dp), lambda i: (i, 0)),
        out_shape=jax.ShapeDtypeStruct((NP, dp), jnp.float32),
    )(h, aa0, aa1, ab0, ab1, degp, b)


def kernel(x, edge_index, W1, b1, W3, b3, W2, b2):
    src = edge_index[0].astype(jnp.int32)
    dst = edge_index[1].astype(jnp.int32)
    pad_e = EP - N_EDGES
    src2d = jnp.concatenate(
        [src, jnp.zeros((pad_e,), jnp.int32)]).reshape(EP // CHUNK, CHUNK)
    dst2d = jnp.concatenate(
        [dst, jnp.full((pad_e,), N_NODES, jnp.int32)]).reshape(EP // CHUNK, CHUNK)
    xp = jnp.pad(x, ((0, NP - N_NODES), (0, 0)))

    degp = _deg_kernel(dst2d).reshape(NW, NP)

    h1, g1 = _tc_pre(xp, W1, degp, 64)
    a1 = _agg64(g1, src2d, dst2d)
    h2, g2 = _tc_mid(h1, a1[0], a1[1], degp, b1.reshape(1, -1), W3, 64)
    a2 = _agg64(g2, src2d, dst2d)
    h3, g3a, g3b = _tc_mid(h2, a2[0], a2[1], degp, b3.reshape(1, -1), W2, 128,
                           split_g=True)
    a3a = _agg64(g3a, src2d, dst2d)
    a3b = _agg64(g3b, src2d, dst2d)
    out = _tc_post(h3, a3a[0], a3a[1], a3b[0], a3b[1], degp, b2.reshape(1, -1))
    return out[:N_NODES]


# speedup vs baseline: 22.6260x; 1.0331x over previous
import functools

import jax
import jax.numpy as jnp
from jax import lax
from jax.experimental import pallas as pl
from jax.experimental.pallas import tpu as pltpu
from jax.experimental.pallas import tpu_sc as plsc

N_NODES = 10000
N_EDGES = 320000
NP = 10240
NC = 2
NS = 16
NW = NC * NS
CHUNK = 128
RPW = 80
EP = NW * RPW * CHUNK
BR = 256

_mesh = plsc.VectorSubcoreMesh(core_axis_name="c", subcore_axis_name="s")
_sc_params = pltpu.CompilerParams(needs_layout_passes=False,
                                  use_tc_tiling_on_sc=False)


@functools.partial(
    pl.kernel,
    out_type=jax.ShapeDtypeStruct((NW * NP,), jnp.float32),
    mesh=_mesh,
    compiler_params=_sc_params,
    scratch_types=[
        pltpu.VMEM((RPW, CHUNK), jnp.int32),
        pltpu.VMEM((NP,), jnp.float32),
    ],
)
def _deg_kernel(dst2d, out, dstv, degl):
    c = lax.axis_index("c")
    s = lax.axis_index("s")
    wid = s * NC + c
    zeros16 = jnp.zeros((16,), jnp.float32)
    ones16 = jnp.ones((16,), jnp.float32)

    def zero_body(i, carry):
        degl[pl.ds(i * 16, 16)] = zeros16
        return carry
    lax.fori_loop(0, NP // 16, zero_body, 0)

    pltpu.sync_copy(dst2d.at[pl.ds(wid * RPW, RPW)], dstv)

    def hist_body(j, carry):
        for l in range(CHUNK // 16):
            idx = dstv[j, pl.ds(l * 16, 16)]
            plsc.addupdate_scatter(degl, [idx], ones16)
        return carry
    lax.fori_loop(0, RPW, hist_body, 0)

    pltpu.sync_copy(degl, out.at[pl.ds(wid * NP, NP)])


RPC = EP // CHUNK // NS


def _make_agg_kernel(d):
    @functools.partial(
        pl.kernel,
        out_type=jax.ShapeDtypeStruct((NC, NP, d), jnp.float32),
        mesh=_mesh,
        compiler_params=_sc_params,
        scratch_types=[
            pltpu.VMEM((RPC, CHUNK), jnp.int32),
            pltpu.VMEM((RPC, CHUNK), jnp.int32),
            pltpu.VMEM((CHUNK, d), jnp.float32),
            pltpu.VMEM((CHUNK, d), jnp.float32),
            pltpu.VMEM_SHARED((NP, d), jnp.float32),
            pltpu.VMEM_SHARED((NP, d), jnp.float32),
            pltpu.SemaphoreType.DMA,
            pltpu.SemaphoreType.DMA,
        ],
    )
    def agg(g_hbm, src2d, dst2d, out, srcv, dstv, buf0, buf1, gshr,
            shared, sem0, sem1):
        c = lax.axis_index("c")
        s = lax.axis_index("s")
        rpt = NP // NS
        zeros16 = jnp.zeros((16,), jnp.float32)

        def zero_body(i, carry):
            for l in range(d // 16):
                buf0[i, pl.ds(l * 16, 16)] = zeros16
            return carry
        lax.fori_loop(0, CHUNK, zero_body, 0)

        pltpu.sync_copy(g_hbm.at[c, pl.ds(s * rpt, rpt)],
                        gshr.at[pl.ds(s * rpt, rpt)])
        for k in range(rpt // CHUNK):
            pltpu.sync_copy(buf0, shared.at[pl.ds(s * rpt + k * CHUNK, CHUNK)])
        plsc.subcore_barrier()

        pltpu.sync_copy(src2d.at[pl.ds(s * RPC, RPC)], srcv)
        pltpu.sync_copy(dst2d.at[pl.ds(s * RPC, RPC)], dstv)

        pltpu.async_copy(gshr.at[srcv.at[0]], buf0, sem0)

        def chunk_body(i, carry):
            base = i * 2
            cp1 = pltpu.async_copy(gshr.at[srcv.at[base + 1]], buf1, sem1)
            pltpu.make_async_copy(gshr.at[srcv.at[base]], buf0, sem0).wait()
            pltpu.sync_copy(buf0, shared.at[dstv.at[base]], add=True)

            @pl.when(base + 2 < RPC)
            def _():
                pltpu.async_copy(gshr.at[srcv.at[base + 2]], buf0, sem0)
            cp1.wait()
            pltpu.sync_copy(buf1, shared.at[dstv.at[base + 1]], add=True)
            return carry
        lax.fori_loop(0, RPC // 2, chunk_body, 0)

        plsc.subcore_barrier()
        for k in range(NP // NS // CHUNK):
            r0 = s * (NP // NS) + k * CHUNK
            pltpu.sync_copy(shared.at[pl.ds(r0, CHUNK)], out.at[c, pl.ds(r0, CHUNK)])

    return agg


_aggh32 = _make_agg_kernel(32)


def _dinv_block(p_block):
    return lax.rsqrt(jnp.sum(p_block, axis=0) + 1.0)[:, None]


def _tc_pre(x, w, degp, dout):
    def body(x_ref, w_ref, p_ref, h_ref, g_ref):
        dinv = _dinv_block(p_ref[...])
        h = jnp.dot(x_ref[...], w_ref[...], preferred_element_type=jnp.float32)
        h_ref[...] = h
        g = h * dinv
        g_ref[0] = g[:, :dout // 2]
        g_ref[1] = g[:, dout // 2:]
    din = x.shape[1]
    return pl.pallas_call(
        body,
        grid=(NP // BR,),
        in_specs=[
            pl.BlockSpec((BR, din), lambda i: (i, 0)),
            pl.BlockSpec((din, dout), lambda i: (0, 0)),
            pl.BlockSpec((NW, BR), lambda i: (0, i)),
        ],
        out_specs=[pl.BlockSpec((BR, dout), lambda i: (i, 0)),
                   pl.BlockSpec((NC, BR, dout // 2), lambda i: (0, i, 0))],
        out_shape=[jax.ShapeDtypeStruct((NP, dout), jnp.float32),
                   jax.ShapeDtypeStruct((NC, NP, dout // 2), jnp.float32)],
    )(x, w, degp)


def _tc_mid(h, a, degp, b, w, dout):
    dp = h.shape[1]
    dh = dp // 2
    n_g = dout // 64
    def body(h_ref, a_ref, p_ref, b_ref, w_ref, hn_ref, *g_refs):
        dinv = _dinv_block(p_ref[...])
        agg = jnp.concatenate([a_ref[0], a_ref[1]], axis=1)
        xn = jnp.maximum(
            dinv * (agg + dinv * h_ref[...]) + b_ref[...], 0.0)
        hn = jnp.dot(xn, w_ref[...], preferred_element_type=jnp.float32)
        hn_ref[...] = hn
        gn = hn * dinv
        for q in range(n_g):
            g_refs[q][0] = gn[:, q * 64:q * 64 + 32]
            g_refs[q][1] = gn[:, q * 64 + 32:q * 64 + 64]
    return pl.pallas_call(
        body,
        grid=(NP // BR,),
        in_specs=[
            pl.BlockSpec((BR, dp), lambda i: (i, 0)),
            pl.BlockSpec((NC, BR, dh), lambda i: (0, i, 0)),
            pl.BlockSpec((NW, BR), lambda i: (0, i)),
            pl.BlockSpec((1, dp), lambda i: (0, 0)),
            pl.BlockSpec((dp, dout), lambda i: (0, 0)),
        ],
        out_specs=[pl.BlockSpec((BR, dout), lambda i: (i, 0))]
        + [pl.BlockSpec((NC, BR, 32), lambda i: (0, i, 0))] * n_g,
        out_shape=[jax.ShapeDtypeStruct((NP, dout), jnp.float32)]
        + [jax.ShapeDtypeStruct((NC, NP, 32), jnp.float32)] * n_g,
    )(h, a, degp, b, w)


def _tc_post(h, a, degp, b):
    dp = h.shape[1]
    dh = dp // 2
    def body(h_ref, a0_ref, a1_ref, p_ref, b_ref, o_ref):
        dinv = _dinv_block(p_ref[...])
        agg = jnp.concatenate(
            [a0_ref[0], a0_ref[1], a1_ref[0], a1_ref[1]], axis=1)
        o_ref[...] = jnp.maximum(
            dinv * (agg + dinv * h_ref[...]) + b_ref[...], 0.0)
    return pl.pallas_call(
        body,
        grid=(NP // BR,),
        in_specs=[
            pl.BlockSpec((BR, dp), lambda i: (i, 0)),
            pl.BlockSpec((NC, BR, 32), lambda i: (0, i, 0)),
            pl.BlockSpec((NC, BR, 32), lambda i: (0, i, 0)),
            pl.BlockSpec((NW, BR), lambda i: (0, i)),
            pl.BlockSpec((1, dp), lambda i: (0, 0)),
        ],
        out_specs=pl.BlockSpec((BR, dp), lambda i: (i, 0)),
        out_shape=jax.ShapeDtypeStruct((NP, dp), jnp.float32),
    )(h, a[0], a[1], degp, b)


def kernel(x, edge_index, W1, b1, W3, b3, W2, b2):
    src = edge_index[0].astype(jnp.int32)
    dst = edge_index[1].astype(jnp.int32)
    pad_e = EP - N_EDGES
    src2d = jnp.concatenate(
        [src, jnp.zeros((pad_e,), jnp.int32)]).reshape(EP // CHUNK, CHUNK)
    dst2d = jnp.concatenate(
        [dst, jnp.full((pad_e,), N_NODES, jnp.int32)]).reshape(EP // CHUNK, CHUNK)
    xp = jnp.pad(x, ((0, NP - N_NODES), (0, 0)))

    degp = _deg_kernel(dst2d).reshape(NW, NP)

    h1, g1 = _tc_pre(xp, W1, degp, 64)
    a1 = _aggh32(g1, src2d, dst2d)
    h2, g2 = _tc_mid(h1, a1, degp, b1.reshape(1, -1), W3, 64)
    a2 = _aggh32(g2, src2d, dst2d)
    h3, g3q0, g3q1 = _tc_mid(h2, a2, degp, b3.reshape(1, -1), W2, 128)
    a3q0 = _aggh32(g3q0, src2d, dst2d)
    a3q1 = _aggh32(g3q1, src2d, dst2d)
    out = _tc_post(h3, (a3q0, a3q1), degp, b2.reshape(1, -1))
    return out[:N_NODES]


# speedup vs baseline: 24.6886x; 1.0912x over previous
import functools

import jax
import jax.numpy as jnp
from jax import lax
from jax.experimental import pallas as pl
from jax.experimental.pallas import tpu as pltpu
from jax.experimental.pallas import tpu_sc as plsc

N_NODES = 10000
N_EDGES = 320000
NP = 10240
NC = 2
NS = 16
NW = NC * NS
CHUNK = 128
RPW = 80
EP = NW * RPW * CHUNK
BR = 512

_mesh = plsc.VectorSubcoreMesh(core_axis_name="c", subcore_axis_name="s")
_sc_params = pltpu.CompilerParams(needs_layout_passes=False,
                                  use_tc_tiling_on_sc=False)


@functools.partial(
    pl.kernel,
    out_type=jax.ShapeDtypeStruct((NW * NP,), jnp.float32),
    mesh=_mesh,
    compiler_params=_sc_params,
    scratch_types=[
        pltpu.VMEM((RPW, CHUNK), jnp.int32),
        pltpu.VMEM((NP,), jnp.float32),
    ],
)
def _deg_kernel(dst2d, out, dstv, degl):
    c = lax.axis_index("c")
    s = lax.axis_index("s")
    wid = s * NC + c
    zeros16 = jnp.zeros((16,), jnp.float32)
    ones16 = jnp.ones((16,), jnp.float32)

    def zero_body(i, carry):
        degl[pl.ds(i * 16, 16)] = zeros16
        return carry
    lax.fori_loop(0, NP // 16, zero_body, 0)

    pltpu.sync_copy(dst2d.at[pl.ds(wid * RPW, RPW)], dstv)

    def hist_body(j, carry):
        for l in range(CHUNK // 16):
            idx = dstv[j, pl.ds(l * 16, 16)]
            plsc.addupdate_scatter(degl, [idx], ones16)
        return carry
    lax.fori_loop(0, RPW, hist_body, 0)

    pltpu.sync_copy(degl, out.at[pl.ds(wid * NP, NP)])


RPC = EP // CHUNK // NS


def _make_agg_kernel(d):
    @functools.partial(
        pl.kernel,
        out_type=jax.ShapeDtypeStruct((NC, NP, d), jnp.float32),
        mesh=_mesh,
        compiler_params=_sc_params,
        scratch_types=[
            pltpu.VMEM((RPC, CHUNK), jnp.int32),
            pltpu.VMEM((RPC, CHUNK), jnp.int32),
            pltpu.VMEM((CHUNK, d), jnp.float32),
            pltpu.VMEM((CHUNK, d), jnp.float32),
            pltpu.VMEM_SHARED((NP, d), jnp.float32),
            pltpu.VMEM_SHARED((NP, d), jnp.float32),
            pltpu.SemaphoreType.DMA,
            pltpu.SemaphoreType.DMA,
        ],
    )
    def agg(g_hbm, src2d, dst2d, out, srcv, dstv, buf0, buf1, gshr,
            shared, sem0, sem1):
        c = lax.axis_index("c")
        s = lax.axis_index("s")
        rpt = NP // NS
        zeros16 = jnp.zeros((16,), jnp.float32)

        def zero_body(i, carry):
            for l in range(d // 16):
                buf0[i, pl.ds(l * 16, 16)] = zeros16
            return carry
        lax.fori_loop(0, CHUNK, zero_body, 0)

        pltpu.sync_copy(g_hbm.at[c, pl.ds(s * rpt, rpt)],
                        gshr.at[pl.ds(s * rpt, rpt)])
        for k in range(rpt // CHUNK):
            pltpu.sync_copy(buf0, shared.at[pl.ds(s * rpt + k * CHUNK, CHUNK)])
        plsc.subcore_barrier()

        pltpu.sync_copy(src2d.at[pl.ds(s * RPC, RPC)], srcv)
        pltpu.sync_copy(dst2d.at[pl.ds(s * RPC, RPC)], dstv)

        pltpu.async_copy(gshr.at[srcv.at[0]], buf0, sem0)

        def chunk_body(i, carry):
            base = i * 2
            cp1 = pltpu.async_copy(gshr.at[srcv.at[base + 1]], buf1, sem1)
            pltpu.make_async_copy(gshr.at[srcv.at[base]], buf0, sem0).wait()
            pltpu.sync_copy(buf0, shared.at[dstv.at[base]], add=True)

            @pl.when(base + 2 < RPC)
            def _():
                pltpu.async_copy(gshr.at[srcv.at[base + 2]], buf0, sem0)
            cp1.wait()
            pltpu.sync_copy(buf1, shared.at[dstv.at[base + 1]], add=True)
            return carry
        lax.fori_loop(0, RPC // 2, chunk_body, 0)

        plsc.subcore_barrier()
        for k in range(NP // NS // CHUNK):
            r0 = s * (NP // NS) + k * CHUNK
            pltpu.sync_copy(shared.at[pl.ds(r0, CHUNK)], out.at[c, pl.ds(r0, CHUNK)])

    return agg


_aggh32 = _make_agg_kernel(32)


def _dinv_block(p_block):
    return lax.rsqrt(jnp.sum(p_block, axis=0) + 1.0)[:, None]


def _tc_pre(x, w, degp, dout):
    def body(x_ref, w_ref, p_ref, h_ref, g_ref):
        dinv = _dinv_block(p_ref[...])
        h = jnp.dot(x_ref[...], w_ref[...], preferred_element_type=jnp.float32)
        h_ref[...] = h
        g = h * dinv
        g_ref[0] = g[:, :dout // 2]
        g_ref[1] = g[:, dout // 2:]
    din = x.shape[1]
    return pl.pallas_call(
        body,
        grid=(NP // BR,),
        in_specs=[
            pl.BlockSpec((BR, din), lambda i: (i, 0)),
            pl.BlockSpec((din, dout), lambda i: (0, 0)),
            pl.BlockSpec((NW, BR), lambda i: (0, i)),
        ],
        out_specs=[pl.BlockSpec((BR, dout), lambda i: (i, 0)),
                   pl.BlockSpec((NC, BR, dout // 2), lambda i: (0, i, 0))],
        out_shape=[jax.ShapeDtypeStruct((NP, dout), jnp.float32),
                   jax.ShapeDtypeStruct((NC, NP, dout // 2), jnp.float32)],
    )(x, w, degp)


def _tc_mid(h, a, degp, b, w, dout):
    dp = h.shape[1]
    dh = dp // 2
    n_g = dout // 64
    def body(h_ref, a_ref, p_ref, b_ref, w_ref, hn_ref, *g_refs):
        dinv = _dinv_block(p_ref[...])
        agg = jnp.concatenate([a_ref[0], a_ref[1]], axis=1)
        xn = jnp.maximum(
            dinv * (agg + dinv * h_ref[...]) + b_ref[...], 0.0)
        hn = jnp.dot(xn, w_ref[...], preferred_element_type=jnp.float32)
        hn_ref[...] = hn
        gn = hn * dinv
        for q in range(n_g):
            g_refs[q][0] = gn[:, q * 64:q * 64 + 32]
            g_refs[q][1] = gn[:, q * 64 + 32:q * 64 + 64]
    return pl.pallas_call(
        body,
        grid=(NP // BR,),
        in_specs=[
            pl.BlockSpec((BR, dp), lambda i: (i, 0)),
            pl.BlockSpec((NC, BR, dh), lambda i: (0, i, 0)),
            pl.BlockSpec((NW, BR), lambda i: (0, i)),
            pl.BlockSpec((1, dp), lambda i: (0, 0)),
            pl.BlockSpec((dp, dout), lambda i: (0, 0)),
        ],
        out_specs=[pl.BlockSpec((BR, dout), lambda i: (i, 0))]
        + [pl.BlockSpec((NC, BR, 32), lambda i: (0, i, 0))] * n_g,
        out_shape=[jax.ShapeDtypeStruct((NP, dout), jnp.float32)]
        + [jax.ShapeDtypeStruct((NC, NP, 32), jnp.float32)] * n_g,
    )(h, a, degp, b, w)


def _tc_post(h, a, degp, b):
    dp = h.shape[1]
    dh = dp // 2
    def body(h_ref, a0_ref, a1_ref, p_ref, b_ref, o_ref):
        dinv = _dinv_block(p_ref[...])
        agg = jnp.concatenate(
            [a0_ref[0], a0_ref[1], a1_ref[0], a1_ref[1]], axis=1)
        o_ref[...] = jnp.maximum(
            dinv * (agg + dinv * h_ref[...]) + b_ref[...], 0.0)
    return pl.pallas_call(
        body,
        grid=(NP // BR,),
        in_specs=[
            pl.BlockSpec((BR, dp), lambda i: (i, 0)),
            pl.BlockSpec((NC, BR, 32), lambda i: (0, i, 0)),
            pl.BlockSpec((NC, BR, 32), lambda i: (0, i, 0)),
            pl.BlockSpec((NW, BR), lambda i: (0, i)),
            pl.BlockSpec((1, dp), lambda i: (0, 0)),
        ],
        out_specs=pl.BlockSpec((BR, dp), lambda i: (i, 0)),
        out_shape=jax.ShapeDtypeStruct((N_NODES, dp), jnp.float32),
    )(h, a[0], a[1], degp, b)


def kernel(x, edge_index, W1, b1, W3, b3, W2, b2):
    src = edge_index[0].astype(jnp.int32)
    dst = edge_index[1].astype(jnp.int32)
    pad_e = EP - N_EDGES
    src2d = jnp.concatenate(
        [src, jnp.zeros((pad_e,), jnp.int32)]).reshape(EP // CHUNK, CHUNK)
    dst2d = jnp.concatenate(
        [dst, jnp.full((pad_e,), N_NODES, jnp.int32)]).reshape(EP // CHUNK, CHUNK)
    degp = _deg_kernel(dst2d).reshape(NW, NP)

    h1, g1 = _tc_pre(x, W1, degp, 64)
    a1 = _aggh32(g1, src2d, dst2d)
    h2, g2 = _tc_mid(h1, a1, degp, b1.reshape(1, -1), W3, 64)
    a2 = _aggh32(g2, src2d, dst2d)
    h3, g3q0, g3q1 = _tc_mid(h2, a2, degp, b3.reshape(1, -1), W2, 128)
    a3q0 = _aggh32(g3q0, src2d, dst2d)
    a3q1 = _aggh32(g3q1, src2d, dst2d)
    return _tc_post(h3, (a3q0, a3q1), degp, b2.reshape(1, -1))


# speedup vs baseline: 25.9621x; 1.0516x over previous
import functools

import jax
import jax.numpy as jnp
from jax import lax
from jax.experimental import pallas as pl
from jax.experimental.pallas import tpu as pltpu
from jax.experimental.pallas import tpu_sc as plsc

N_NODES = 10000
N_EDGES = 320000
NP = 10240
NC = 2
NS = 16
NW = NC * NS
CHUNK = 128
RPW = 80
EP = NW * RPW * CHUNK
BR = 1024

_mesh = plsc.VectorSubcoreMesh(core_axis_name="c", subcore_axis_name="s")
_sc_params = pltpu.CompilerParams(needs_layout_passes=False,
                                  use_tc_tiling_on_sc=False)


@functools.partial(
    pl.kernel,
    out_type=jax.ShapeDtypeStruct((NW * NP,), jnp.float32),
    mesh=_mesh,
    compiler_params=_sc_params,
    scratch_types=[
        pltpu.VMEM((RPW, CHUNK), jnp.int32),
        pltpu.VMEM((NP,), jnp.float32),
    ],
)
def _deg_kernel(dst2d, out, dstv, degl):
    c = lax.axis_index("c")
    s = lax.axis_index("s")
    wid = s * NC + c
    zeros16 = jnp.zeros((16,), jnp.float32)
    ones16 = jnp.ones((16,), jnp.float32)

    def zero_body(i, carry):
        degl[pl.ds(i * 16, 16)] = zeros16
        return carry
    lax.fori_loop(0, NP // 16, zero_body, 0)

    pltpu.sync_copy(dst2d.at[pl.ds(wid * RPW, RPW)], dstv)

    def hist_body(j, carry):
        for l in range(CHUNK // 16):
            idx = dstv[j, pl.ds(l * 16, 16)]
            plsc.addupdate_scatter(degl, [idx], ones16)
        return carry
    lax.fori_loop(0, RPW, hist_body, 0)

    pltpu.sync_copy(degl, out.at[pl.ds(wid * NP, NP)])


RPC = EP // CHUNK // NS


def _make_agg_kernel(d):
    @functools.partial(
        pl.kernel,
        out_type=jax.ShapeDtypeStruct((NC, NP, d), jnp.float32),
        mesh=_mesh,
        compiler_params=_sc_params,
        scratch_types=[
            pltpu.VMEM((RPC, CHUNK), jnp.int32),
            pltpu.VMEM((RPC, CHUNK), jnp.int32),
            pltpu.VMEM((CHUNK, d), jnp.float32),
            pltpu.VMEM((CHUNK, d), jnp.float32),
            pltpu.VMEM_SHARED((NP, d), jnp.float32),
            pltpu.VMEM_SHARED((NP, d), jnp.float32),
            pltpu.SemaphoreType.DMA,
            pltpu.SemaphoreType.DMA,
        ],
    )
    def agg(g_hbm, src2d, dst2d, out, srcv, dstv, buf0, buf1, gshr,
            shared, sem0, sem1):
        c = lax.axis_index("c")
        s = lax.axis_index("s")
        rpt = NP // NS
        zeros16 = jnp.zeros((16,), jnp.float32)

        def zero_body(i, carry):
            for l in range(d // 16):
                buf0[i, pl.ds(l * 16, 16)] = zeros16
            return carry
        lax.fori_loop(0, CHUNK, zero_body, 0)

        pltpu.sync_copy(g_hbm.at[c, pl.ds(s * rpt, rpt)],
                        gshr.at[pl.ds(s * rpt, rpt)])
        for k in range(rpt // CHUNK):
            pltpu.sync_copy(buf0, shared.at[pl.ds(s * rpt + k * CHUNK, CHUNK)])
        plsc.subcore_barrier()

        pltpu.sync_copy(src2d.at[pl.ds(s * RPC, RPC)], srcv)
        pltpu.sync_copy(dst2d.at[pl.ds(s * RPC, RPC)], dstv)

        pltpu.async_copy(gshr.at[srcv.at[0]], buf0, sem0)

        def chunk_body(i, carry):
            base = i * 2
            cp1 = pltpu.async_copy(gshr.at[srcv.at[base + 1]], buf1, sem1)
            pltpu.make_async_copy(gshr.at[srcv.at[base]], buf0, sem0).wait()
            pltpu.sync_copy(buf0, shared.at[dstv.at[base]], add=True)

            @pl.when(base + 2 < RPC)
            def _():
                pltpu.async_copy(gshr.at[srcv.at[base + 2]], buf0, sem0)
            cp1.wait()
            pltpu.sync_copy(buf1, shared.at[dstv.at[base + 1]], add=True)
            return carry
        lax.fori_loop(0, RPC // 2, chunk_body, 0)

        plsc.subcore_barrier()
        for k in range(NP // NS // CHUNK):
            r0 = s * (NP // NS) + k * CHUNK
            pltpu.sync_copy(shared.at[pl.ds(r0, CHUNK)], out.at[c, pl.ds(r0, CHUNK)])

    return agg


_aggh32 = _make_agg_kernel(32)


def _dinv_block(p_block):
    return lax.rsqrt(jnp.sum(p_block, axis=0) + 1.0)[:, None]


def _tc_pre(x, w, degp, dout):
    def body(x_ref, w_ref, p_ref, h_ref, g_ref):
        dinv = _dinv_block(p_ref[...])
        h = jnp.dot(x_ref[...], w_ref[...], preferred_element_type=jnp.float32)
        h_ref[...] = h
        g = h * dinv
        g_ref[0] = g[:, :dout // 2]
        g_ref[1] = g[:, dout // 2:]
    din = x.shape[1]
    return pl.pallas_call(
        body,
        grid=(NP // BR,),
        in_specs=[
            pl.BlockSpec((BR, din), lambda i: (i, 0)),
            pl.BlockSpec((din, dout), lambda i: (0, 0)),
            pl.BlockSpec((NW, BR), lambda i: (0, i)),
        ],
        out_specs=[pl.BlockSpec((BR, dout), lambda i: (i, 0)),
                   pl.BlockSpec((NC, BR, dout // 2), lambda i: (0, i, 0))],
        out_shape=[jax.ShapeDtypeStruct((NP, dout), jnp.float32),
                   jax.ShapeDtypeStruct((NC, NP, dout // 2), jnp.float32)],
    )(x, w, degp)


def _tc_mid(h, a, degp, b, w, dout):
    dp = h.shape[1]
    dh = dp // 2
    n_g = dout // 64
    def body(h_ref, a_ref, p_ref, b_ref, w_ref, hn_ref, *g_refs):
        dinv = _dinv_block(p_ref[...])
        agg = jnp.concatenate([a_ref[0], a_ref[1]], axis=1)
        xn = jnp.maximum(
            dinv * (agg + dinv * h_ref[...]) + b_ref[...], 0.0)
        hn = jnp.dot(xn, w_ref[...], preferred_element_type=jnp.float32)
        hn_ref[...] = hn
        gn = hn * dinv
        for q in range(n_g):
            g_refs[q][0] = gn[:, q * 64:q * 64 + 32]
            g_refs[q][1] = gn[:, q * 64 + 32:q * 64 + 64]
    return pl.pallas_call(
        body,
        grid=(NP // BR,),
        in_specs=[
            pl.BlockSpec((BR, dp), lambda i: (i, 0)),
            pl.BlockSpec((NC, BR, dh), lambda i: (0, i, 0)),
            pl.BlockSpec((NW, BR), lambda i: (0, i)),
            pl.BlockSpec((1, dp), lambda i: (0, 0)),
            pl.BlockSpec((dp, dout), lambda i: (0, 0)),
        ],
        out_specs=[pl.BlockSpec((BR, dout), lambda i: (i, 0))]
        + [pl.BlockSpec((NC, BR, 32), lambda i: (0, i, 0))] * n_g,
        out_shape=[jax.ShapeDtypeStruct((NP, dout), jnp.float32)]
        + [jax.ShapeDtypeStruct((NC, NP, 32), jnp.float32)] * n_g,
    )(h, a, degp, b, w)


def _tc_post(h, a, degp, b):
    dp = h.shape[1]
    dh = dp // 2
    def body(h_ref, a0_ref, a1_ref, p_ref, b_ref, o_ref):
        dinv = _dinv_block(p_ref[...])
        agg = jnp.concatenate(
            [a0_ref[0], a0_ref[1], a1_ref[0], a1_ref[1]], axis=1)
        o_ref[...] = jnp.maximum(
            dinv * (agg + dinv * h_ref[...]) + b_ref[...], 0.0)
    return pl.pallas_call(
        body,
        grid=(NP // BR,),
        in_specs=[
            pl.BlockSpec((BR, dp), lambda i: (i, 0)),
            pl.BlockSpec((NC, BR, 32), lambda i: (0, i, 0)),
            pl.BlockSpec((NC, BR, 32), lambda i: (0, i, 0)),
            pl.BlockSpec((NW, BR), lambda i: (0, i)),
            pl.BlockSpec((1, dp), lambda i: (0, 0)),
        ],
        out_specs=pl.BlockSpec((BR, dp), lambda i: (i, 0)),
        out_shape=jax.ShapeDtypeStruct((N_NODES, dp), jnp.float32),
    )(h, a[0], a[1], degp, b)


def kernel(x, edge_index, W1, b1, W3, b3, W2, b2):
    src = edge_index[0].astype(jnp.int32)
    dst = edge_index[1].astype(jnp.int32)
    pad_e = EP - N_EDGES
    src2d = jnp.concatenate(
        [src, jnp.zeros((pad_e,), jnp.int32)]).reshape(EP // CHUNK, CHUNK)
    dst2d = jnp.concatenate(
        [dst, jnp.full((pad_e,), N_NODES, jnp.int32)]).reshape(EP // CHUNK, CHUNK)
    degp = _deg_kernel(dst2d).reshape(NW, NP)

    h1, g1 = _tc_pre(x, W1, degp, 64)
    a1 = _aggh32(g1, src2d, dst2d)
    h2, g2 = _tc_mid(h1, a1, degp, b1.reshape(1, -1), W3, 64)
    a2 = _aggh32(g2, src2d, dst2d)
    h3, g3q0, g3q1 = _tc_mid(h2, a2, degp, b3.reshape(1, -1), W2, 128)
    a3q0 = _aggh32(g3q0, src2d, dst2d)
    a3q1 = _aggh32(g3q1, src2d, dst2d)
    return _tc_post(h3, (a3q0, a3q1), degp, b2.reshape(1, -1))
